# baseline (speedup 1.0000x reference)
import functools

import jax
import jax.numpy as jnp
from jax import lax
from jax.experimental import pallas as pl
from jax.experimental.pallas import tpu as pltpu

S, B, I = 50, 256, 200
E, M = 80, 256
T = 50
T_TOT = S * T
A_I, A_V = 0.9, 0.9
ETA = 0.01
LAM = 0.951229424500714

WIN_LO, WIN_HI = 19, 48


def _spike(v):
    return 0.5 + 0.5 * jnp.tanh(5.0 * (v - 1.0))


def _dotT(x, w):
    return lax.dot_general(x, w, (((x.ndim - 1,), (1,)), ((), ())))


BLK_A = 64


def _chains_body(facts_ref, wemb_ref, wenc_ref, wcat_ref,
                 tk_out, zv_out, zrk_out,
                 ze_s, xin_s,
                 enc_s, st_s, tk_s):
    c = pl.program_id(1)

    @pl.when(c == 0)
    def _():
        enc_s[...] = jnp.zeros_like(enc_s)
        st_s[...] = jnp.zeros_like(st_s)
        tk_s[...] = jnp.zeros_like(tk_s)

    x = facts_ref[0]
    emb = _dotT(x, wemb_ref[...])
    enc_in = _dotT(emb, wenc_ref[...])

    def p1(j, st):
        i, v, th = st
        i = A_I * i + enc_in
        v = A_V * v * (0.5 * (1.0 - th)) + i
        th = jnp.tanh(5.0 * v - 5.0)
        ze_s[pl.ds(j * BLK_A, BLK_A), :] = 0.5 + 0.5 * th
        return (i, v, th)

    enc_st = lax.fori_loop(
        0, T, p1, (enc_s[0], enc_s[1], enc_s[2]), unroll=5)
    enc_s[0] = enc_st[0]
    enc_s[1] = enc_st[1]
    enc_s[2] = enc_st[2]

    xin_s[...] = _dotT(ze_s[...], wcat_ref[...])

    def chain(lane0, out_ref, with_trace):
        def body(j, st):
            i, v, th, tk = st
            i = A_I * i + xin_s[pl.ds(j * BLK_A, BLK_A), lane0:lane0 + M]
            v = A_V * v * (0.5 * (1.0 - th)) + i
            th = jnp.tanh(5.0 * v - 5.0)
            z = 0.5 + 0.5 * th
            if with_trace:
                tk = LAM * tk + z
                out_ref[j] = tk.astype(jnp.bfloat16)
            else:
                out_ref[j] = z.astype(jnp.bfloat16)
            return (i, v, th, tk)

        k = lane0 // M
        st0 = (st_s[0, :, lane0:lane0 + M], st_s[1, :, lane0:lane0 + M],
               st_s[2, :, lane0:lane0 + M], tk_s[...])
        stf = lax.fori_loop(0, T, body, st0, unroll=5)
        st_s[0, :, lane0:lane0 + M] = stf[0]
        st_s[1, :, lane0:lane0 + M] = stf[1]
        st_s[2, :, lane0:lane0 + M] = stf[2]
        if with_trace:
            tk_s[...] = stf[3]

    chain(0, tk_out, True)
    chain(M, zv_out, False)
    chain(2 * M, zrk_out, False)


BLK_B = 32
CH_B = 125


def _memory_body(tk_ref, zv_ref, zrk_ref, mem0_ref, rv_out,
                 mem_s, tkT, zvT, zrkT, rvT):
    c = pl.program_id(1)

    @pl.when(c == 0)
    def _():
        mem_s[...] = mem0_ref[...]

    tkT[...] = jnp.swapaxes(tk_ref[...], 0, 1)
    zvT[...] = jnp.swapaxes(zv_ref[...], 0, 1)
    zrkT[...] = jnp.swapaxes(zrk_ref[...], 0, 1)

    row = lax.broadcasted_iota(jnp.int32, (CH_B, CH_B), 0)
    col = lax.broadcasted_iota(jnp.int32, (CH_B, CH_B), 1)
    causal = col <= row

    for be in range(BLK_B):
        tkb = tkT[be]
        zvb = zvT[be]
        zrkb = zrkT[be]
        memb = mem_s[be]
        g = lax.dot_general(zrkb, tkb, (((1,), (1,)), ((), ())),
                            preferred_element_type=jnp.float32)
        gm = jnp.where(causal, g, 0.0).astype(jnp.bfloat16)
        intra = lax.dot_general(gm, zvb, (((1,), (0,)), ((), ())),
                                preferred_element_type=jnp.float32)
        inter = lax.dot_general(zrkb, memb.astype(jnp.bfloat16),
                                (((1,), (1,)), ((), ())),
                                preferred_element_type=jnp.float32)
        rvT[be] = (inter + ETA * intra).astype(jnp.bfloat16)
        mem_s[be] = memb + ETA * lax.dot_general(
            zvb, tkb, (((0,), (0,)), ((), ())),
            preferred_element_type=jnp.float32)

    rv_out[...] = jnp.swapaxes(rvT[...], 0, 1)


BLK_C = 64


def _readout_body(rv_ref, out_ref, st_s, acc_s):
    c = pl.program_id(1)

    @pl.when(c == 0)
    def _():
        st_s[...] = jnp.zeros_like(st_s)

    acc_s[...] = jnp.zeros_like(acc_s)

    def step(j, st, accumulate):
        i, v, th = st
        i = A_I * i + rv_ref[j]
        v = A_V * v * (0.5 * (1.0 - th)) + i
        th = jnp.tanh(5.0 * v - 5.0)
        if accumulate:
            acc_s[...] = acc_s[...] + (0.5 + 0.5 * th)
        return (i, v, th)

    st = (st_s[0], st_s[1], st_s[2])
    st = lax.fori_loop(0, WIN_LO, functools.partial(step, accumulate=False),
                       st, unroll=5)
    st = lax.fori_loop(WIN_LO, WIN_HI + 1,
                       functools.partial(step, accumulate=True), st, unroll=5)
    st = lax.fori_loop(WIN_HI + 1, T, functools.partial(step, accumulate=False),
                       st, unroll=5)
    st_s[0] = st[0]
    st_s[1] = st[1]
    st_s[2] = st[2]

    out_ref[0] = acc_s[...]


def kernel(facts, mem0, W_emb, W_enc, W_wk, W_wv, W_rk):
    f32 = jnp.float32
    bf16 = jnp.bfloat16
    zseq = jax.ShapeDtypeStruct((T_TOT, B, M), bf16)
    W_cat = jnp.concatenate([W_wk, W_wv, W_rk], axis=0)

    tk_all, zv_all, zrk_all = pl.pallas_call(
        _chains_body,
        out_shape=(zseq, zseq, zseq),
        grid=(B // BLK_A, S),
        in_specs=[
            pl.BlockSpec((1, BLK_A, I), lambda b, c: (c, b, 0)),
            pl.BlockSpec((E, I), lambda b, c: (0, 0)),
            pl.BlockSpec((E, E), lambda b, c: (0, 0)),
            pl.BlockSpec((3 * M, E), lambda b, c: (0, 0)),
        ],
        out_specs=(
            pl.BlockSpec((T, BLK_A, M), lambda b, c: (c, b, 0)),
            pl.BlockSpec((T, BLK_A, M), lambda b, c: (c, b, 0)),
            pl.BlockSpec((T, BLK_A, M), lambda b, c: (c, b, 0)),
        ),
        scratch_shapes=[
            pltpu.VMEM((T * BLK_A, E), f32),
            pltpu.VMEM((T * BLK_A, 3 * M), f32),
            pltpu.VMEM((3, BLK_A, E), f32),
            pltpu.VMEM((3, BLK_A, 3 * M), f32),
            pltpu.VMEM((BLK_A, M), f32),
        ],
        compiler_params=pltpu.CompilerParams(
            dimension_semantics=("parallel", "arbitrary"),
            vmem_limit_bytes=100 * 1024 * 1024,
        ),
        name="lif_chains",
    )(facts, W_emb, W_enc, W_cat)

    rv_all = pl.pallas_call(
        _memory_body,
        out_shape=zseq,
        grid=(B // BLK_B, T_TOT // CH_B),
        in_specs=[
            pl.BlockSpec((CH_B, BLK_B, M), lambda b, c: (c, b, 0)),
            pl.BlockSpec((CH_B, BLK_B, M), lambda b, c: (c, b, 0)),
            pl.BlockSpec((CH_B, BLK_B, M), lambda b, c: (c, b, 0)),
            pl.BlockSpec((BLK_B, M, M), lambda b, c: (b, 0, 0)),
        ],
        out_specs=pl.BlockSpec((CH_B, BLK_B, M), lambda b, c: (c, b, 0)),
        scratch_shapes=[
            pltpu.VMEM((BLK_B, M, M), f32),
            pltpu.VMEM((BLK_B, CH_B, M), bf16),
            pltpu.VMEM((BLK_B, CH_B, M), bf16),
            pltpu.VMEM((BLK_B, CH_B, M), bf16),
            pltpu.VMEM((BLK_B, CH_B, M), bf16),
        ],
        compiler_params=pltpu.CompilerParams(
            dimension_semantics=("parallel", "arbitrary"),
            vmem_limit_bytes=100 * 1024 * 1024,
        ),
        name="memory_read",
    )(tk_all, zv_all, zrk_all, mem0)

    out = pl.pallas_call(
        _readout_body,
        out_shape=jax.ShapeDtypeStruct((S, B, M), f32),
        grid=(B // BLK_C, S),
        in_specs=[pl.BlockSpec((T, BLK_C, M), lambda b, c: (c, b, 0))],
        out_specs=pl.BlockSpec((1, BLK_C, M), lambda b, c: (c, b, 0)),
        scratch_shapes=[pltpu.VMEM((3, BLK_C, M), f32),
                        pltpu.VMEM((BLK_C, M), f32)],
        compiler_params=pltpu.CompilerParams(
            dimension_semantics=("parallel", "arbitrary"),
            vmem_limit_bytes=100 * 1024 * 1024,
        ),
        name="readout",
    )(rv_all)

    return out


# speedup vs baseline: 1.0015x; 1.0015x over previous
import functools

import jax
import jax.numpy as jnp
from jax import lax
from jax.experimental import pallas as pl
from jax.experimental.pallas import tpu as pltpu

S, B, I = 50, 256, 200
E, M = 80, 256
T = 50
T_TOT = S * T
A_I, A_V = 0.9, 0.9
ETA = 0.01
LAM = 0.951229424500714

WIN_LO, WIN_HI = 19, 48


def _dotT(x, w):
    return lax.dot_general(x, w, (((x.ndim - 1,), (1,)), ((), ())))


BLK_A = 64


def _chains_body(facts_ref, wemb_ref, wenc_ref, wcat_ref,
                 tk_out, zv_out, zrk_out,
                 ze_s, xin_s,
                 enc_s, st_s, tk_s):
    c = pl.program_id(1)

    @pl.when(c == 0)
    def _():
        enc_s[...] = jnp.zeros_like(enc_s)
        st_s[...] = jnp.zeros_like(st_s)
        tk_s[...] = jnp.zeros_like(tk_s)

    x = facts_ref[0]
    emb = _dotT(x, wemb_ref[...])
    enc_in = _dotT(emb, wenc_ref[...])

    def p1(j, st):
        i, v, th = st
        i = A_I * i + enc_in
        v = A_V * v * (0.5 * (1.0 - th)) + i
        th = jnp.tanh(5.0 * v - 5.0)
        ze_s[pl.ds(j * BLK_A, BLK_A), :] = 0.5 + 0.5 * th
        return (i, v, th)

    enc_st = lax.fori_loop(
        0, T, p1, (enc_s[0], enc_s[1], enc_s[2]), unroll=5)
    enc_s[0] = enc_st[0]
    enc_s[1] = enc_st[1]
    enc_s[2] = enc_st[2]

    xin_s[...] = _dotT(ze_s[...], wcat_ref[...])

    def chain(lane0, out_ref, with_trace):
        def body(j, st):
            i, v, th, tk = st
            i = A_I * i + xin_s[pl.ds(j * BLK_A, BLK_A), lane0:lane0 + M]
            v = A_V * v * (0.5 * (1.0 - th)) + i
            th = jnp.tanh(5.0 * v - 5.0)
            z = 0.5 + 0.5 * th
            if with_trace:
                tk = LAM * tk + z
                out_ref[j] = tk.astype(jnp.bfloat16)
            else:
                out_ref[j] = z.astype(jnp.bfloat16)
            return (i, v, th, tk)

        st0 = (st_s[0, :, lane0:lane0 + M], st_s[1, :, lane0:lane0 + M],
               st_s[2, :, lane0:lane0 + M], tk_s[...])
        stf = lax.fori_loop(0, T, body, st0, unroll=5)
        st_s[0, :, lane0:lane0 + M] = stf[0]
        st_s[1, :, lane0:lane0 + M] = stf[1]
        st_s[2, :, lane0:lane0 + M] = stf[2]
        if with_trace:
            tk_s[...] = stf[3]

    chain(0, tk_out, True)
    chain(M, zv_out, False)
    chain(2 * M, zrk_out, False)


BLK_B = 32
CH_B = 125


def _memory_body(tk_ref, zv_ref, zrk_ref, mem0_ref, rv_out,
                 mem_s, tkT, zvT, zrkT, rvT):
    c = pl.program_id(1)

    @pl.when(c == 0)
    def _():
        mem_s[...] = mem0_ref[...]

    tkT[...] = jnp.swapaxes(tk_ref[...], 0, 1)
    zvT[...] = jnp.swapaxes(zv_ref[...], 0, 1)
    zrkT[...] = jnp.swapaxes(zrk_ref[...], 0, 1)

    row = lax.broadcasted_iota(jnp.int32, (CH_B, CH_B), 0)
    col = lax.broadcasted_iota(jnp.int32, (CH_B, CH_B), 1)
    causal = col <= row

    for be in range(BLK_B):
        tkb = tkT[be]
        zvb = zvT[be]
        zrkb = zrkT[be]
        memb = mem_s[be]
        g = lax.dot_general(zrkb, tkb, (((1,), (1,)), ((), ())),
                            preferred_element_type=jnp.float32)
        gm = jnp.where(causal, g, 0.0).astype(jnp.bfloat16)
        intra = lax.dot_general(gm, zvb, (((1,), (0,)), ((), ())),
                                preferred_element_type=jnp.float32)
        inter = lax.dot_general(zrkb, memb.astype(jnp.bfloat16),
                                (((1,), (1,)), ((), ())),
                                preferred_element_type=jnp.float32)
        rvT[be] = (inter + ETA * intra).astype(jnp.bfloat16)
        mem_s[be] = memb + ETA * lax.dot_general(
            zvb, tkb, (((0,), (0,)), ((), ())),
            preferred_element_type=jnp.float32)

    rv_out[...] = jnp.swapaxes(rvT[...], 0, 1)


BLK_C = 64


def _readout_body(rv_ref, out_ref, st_s, acc_s):
    c = pl.program_id(1)

    @pl.when(c == 0)
    def _():
        st_s[...] = jnp.zeros_like(st_s)

    acc_s[...] = jnp.zeros_like(acc_s)

    def step(j, st, accumulate):
        i, v, th = st
        i = A_I * i + rv_ref[j]
        v = A_V * v * (0.5 * (1.0 - th)) + i
        th = jnp.tanh(5.0 * v - 5.0)
        if accumulate:
            acc_s[...] = acc_s[...] + (0.5 + 0.5 * th)
        return (i, v, th)

    st = (st_s[0], st_s[1], st_s[2])
    st = lax.fori_loop(0, WIN_LO, functools.partial(step, accumulate=False),
                       st, unroll=5)
    st = lax.fori_loop(WIN_LO, WIN_HI + 1,
                       functools.partial(step, accumulate=True), st, unroll=5)
    st = lax.fori_loop(WIN_HI + 1, T, functools.partial(step, accumulate=False),
                       st, unroll=5)
    st_s[0] = st[0]
    st_s[1] = st[1]
    st_s[2] = st[2]

    out_ref[0] = acc_s[...]


def kernel(facts, mem0, W_emb, W_enc, W_wk, W_wv, W_rk):
    f32 = jnp.float32
    bf16 = jnp.bfloat16
    zseq = jax.ShapeDtypeStruct((T_TOT, B, M), bf16)
    W_cat = jnp.concatenate([W_wk, W_wv, W_rk], axis=0)

    tk_all, zv_all, zrk_all = pl.pallas_call(
        _chains_body,
        out_shape=(zseq, zseq, zseq),
        grid=(B // BLK_A, S),
        in_specs=[
            pl.BlockSpec((1, BLK_A, I), lambda b, c: (c, b, 0)),
            pl.BlockSpec((E, I), lambda b, c: (0, 0)),
            pl.BlockSpec((E, E), lambda b, c: (0, 0)),
            pl.BlockSpec((3 * M, E), lambda b, c: (0, 0)),
        ],
        out_specs=(
            pl.BlockSpec((T, BLK_A, M), lambda b, c: (c, b, 0)),
            pl.BlockSpec((T, BLK_A, M), lambda b, c: (c, b, 0)),
            pl.BlockSpec((T, BLK_A, M), lambda b, c: (c, b, 0)),
        ),
        scratch_shapes=[
            pltpu.VMEM((T * BLK_A, E), f32),
            pltpu.VMEM((T * BLK_A, 3 * M), f32),
            pltpu.VMEM((3, BLK_A, E), f32),
            pltpu.VMEM((3, BLK_A, 3 * M), f32),
            pltpu.VMEM((BLK_A, M), f32),
        ],
        compiler_params=pltpu.CompilerParams(
            dimension_semantics=("parallel", "arbitrary"),
            vmem_limit_bytes=100 * 1024 * 1024,
        ),
        name="lif_chains",
    )(facts, W_emb, W_enc, W_cat)

    rv_all = pl.pallas_call(
        _memory_body,
        out_shape=zseq,
        grid=(B // BLK_B, T_TOT // CH_B),
        in_specs=[
            pl.BlockSpec((CH_B, BLK_B, M), lambda b, c: (c, b, 0)),
            pl.BlockSpec((CH_B, BLK_B, M), lambda b, c: (c, b, 0)),
            pl.BlockSpec((CH_B, BLK_B, M), lambda b, c: (c, b, 0)),
            pl.BlockSpec((BLK_B, M, M), lambda b, c: (b, 0, 0)),
        ],
        out_specs=pl.BlockSpec((CH_B, BLK_B, M), lambda b, c: (c, b, 0)),
        scratch_shapes=[
            pltpu.VMEM((BLK_B, M, M), f32),
            pltpu.VMEM((BLK_B, CH_B, M), bf16),
            pltpu.VMEM((BLK_B, CH_B, M), bf16),
            pltpu.VMEM((BLK_B, CH_B, M), bf16),
            pltpu.VMEM((BLK_B, CH_B, M), bf16),
        ],
        compiler_params=pltpu.CompilerParams(
            dimension_semantics=("parallel", "arbitrary"),
            vmem_limit_bytes=100 * 1024 * 1024,
        ),
        name="memory_read",
    )(tk_all, zv_all, zrk_all, mem0)

    out = pl.pallas_call(
        _readout_body,
        out_shape=jax.ShapeDtypeStruct((S, B, M), f32),
        grid=(B // BLK_C, S),
        in_specs=[pl.BlockSpec((T, BLK_C, M), lambda b, c: (c, b, 0))],
        out_specs=pl.BlockSpec((1, BLK_C, M), lambda b, c: (c, b, 0)),
        scratch_shapes=[pltpu.VMEM((3, BLK_C, M), f32),
                        pltpu.VMEM((BLK_C, M), f32)],
        compiler_params=pltpu.CompilerParams(
            dimension_semantics=("parallel", "arbitrary"),
            vmem_limit_bytes=100 * 1024 * 1024,
        ),
        name="readout",
    )(rv_all)

    return out


# speedup vs baseline: 1.0493x; 1.0477x over previous
import functools

import jax
import jax.numpy as jnp
from jax import lax
from jax.experimental import pallas as pl
from jax.experimental.pallas import tpu as pltpu

S, B, I = 50, 256, 200
E, M = 80, 256
T = 50
T_TOT = S * T
A_I, A_V = 0.9, 0.9
ETA = 0.01
LAM = 0.951229424500714

WIN_LO, WIN_HI = 19, 48


def _dotT(x, w):
    return lax.dot_general(x, w, (((x.ndim - 1,), (1,)), ((), ())))


BLK_A = 64


def _chains_body(facts_ref, wemb_ref, wenc_ref, wcat_ref,
                 tk_out, zv_out, zrk_out,
                 ze_s, xin_s,
                 enc_s, st_s, tk_s):
    c = pl.program_id(1)

    @pl.when(c == 0)
    def _():
        enc_s[...] = jnp.zeros_like(enc_s)
        st_s[...] = jnp.zeros_like(st_s)
        tk_s[...] = jnp.zeros_like(tk_s)

    x = facts_ref[0]
    emb = _dotT(x, wemb_ref[...])
    enc_in = _dotT(emb, wenc_ref[...])

    def p1(j, st):
        i, v, th = st
        i = A_I * i + enc_in
        v = A_V * v * (0.5 * (1.0 - th)) + i
        th = jnp.tanh(5.0 * v - 5.0)
        ze_s[pl.ds(j * BLK_A, BLK_A), :] = 0.5 + 0.5 * th
        return (i, v, th)

    enc_st = lax.fori_loop(
        0, T, p1, (enc_s[0], enc_s[1], enc_s[2]), unroll=5)
    enc_s[0] = enc_st[0]
    enc_s[1] = enc_st[1]
    enc_s[2] = enc_st[2]

    xin_s[...] = _dotT(ze_s[...], wcat_ref[...])

    def chain(lane0, out_ref, with_trace):
        def body(j, st):
            i, v, th, tk = st
            i = A_I * i + xin_s[pl.ds(j * BLK_A, BLK_A), lane0:lane0 + M]
            v = A_V * v * (0.5 * (1.0 - th)) + i
            th = jnp.tanh(5.0 * v - 5.0)
            z = 0.5 + 0.5 * th
            if with_trace:
                tk = LAM * tk + z
                out_ref[j] = tk.astype(jnp.bfloat16)
            else:
                out_ref[j] = z.astype(jnp.bfloat16)
            return (i, v, th, tk)

        st0 = (st_s[0, :, lane0:lane0 + M], st_s[1, :, lane0:lane0 + M],
               st_s[2, :, lane0:lane0 + M], tk_s[...])
        stf = lax.fori_loop(0, T, body, st0, unroll=10)
        st_s[0, :, lane0:lane0 + M] = stf[0]
        st_s[1, :, lane0:lane0 + M] = stf[1]
        st_s[2, :, lane0:lane0 + M] = stf[2]
        if with_trace:
            tk_s[...] = stf[3]

    chain(0, tk_out, True)
    chain(M, zv_out, False)
    chain(2 * M, zrk_out, False)


BLK_B = 32
CH_B = 125


def _memory_body(tk_ref, zv_ref, zrk_ref, mem0_ref, rv_out,
                 mem_s, tkT, zvT, zrkT, rvT):
    c = pl.program_id(1)

    @pl.when(c == 0)
    def _():
        mem_s[...] = mem0_ref[...]

    tkT[...] = jnp.swapaxes(tk_ref[...], 0, 1)
    zvT[...] = jnp.swapaxes(zv_ref[...], 0, 1)
    zrkT[...] = jnp.swapaxes(zrk_ref[...], 0, 1)

    row = lax.broadcasted_iota(jnp.int32, (CH_B, CH_B), 0)
    col = lax.broadcasted_iota(jnp.int32, (CH_B, CH_B), 1)
    causal = col <= row

    for be in range(BLK_B):
        tkb = tkT[be]
        zvb = zvT[be]
        zrkb = zrkT[be]
        memb = mem_s[be]
        g = lax.dot_general(zrkb, tkb, (((1,), (1,)), ((), ())),
                            preferred_element_type=jnp.float32)
        gm = jnp.where(causal, g, 0.0).astype(jnp.bfloat16)
        intra = lax.dot_general(gm, zvb, (((1,), (0,)), ((), ())),
                                preferred_element_type=jnp.float32)
        inter = lax.dot_general(zrkb, memb.astype(jnp.bfloat16),
                                (((1,), (1,)), ((), ())),
                                preferred_element_type=jnp.float32)
        rvT[be] = (inter + ETA * intra).astype(jnp.bfloat16)
        mem_s[be] = memb + ETA * lax.dot_general(
            zvb, tkb, (((0,), (0,)), ((), ())),
            preferred_element_type=jnp.float32)

    rv_out[...] = jnp.swapaxes(rvT[...], 0, 1)


BLK_C = 64


def _readout_body(rv_ref, out_ref, st_s, acc_s):
    c = pl.program_id(1)

    @pl.when(c == 0)
    def _():
        st_s[...] = jnp.zeros_like(st_s)

    acc_s[...] = jnp.zeros_like(acc_s)

    def step(j, st, accumulate):
        i, v, th = st
        i = A_I * i + rv_ref[j]
        v = A_V * v * (0.5 * (1.0 - th)) + i
        th = jnp.tanh(5.0 * v - 5.0)
        if accumulate:
            acc_s[...] = acc_s[...] + (0.5 + 0.5 * th)
        return (i, v, th)

    st = (st_s[0], st_s[1], st_s[2])
    st = lax.fori_loop(0, WIN_LO, functools.partial(step, accumulate=False),
                       st, unroll=5)
    st = lax.fori_loop(WIN_LO, WIN_HI + 1,
                       functools.partial(step, accumulate=True), st, unroll=5)
    st = lax.fori_loop(WIN_HI + 1, T, functools.partial(step, accumulate=False),
                       st, unroll=5)
    st_s[0] = st[0]
    st_s[1] = st[1]
    st_s[2] = st[2]

    out_ref[0] = acc_s[...]


def kernel(facts, mem0, W_emb, W_enc, W_wk, W_wv, W_rk):
    f32 = jnp.float32
    bf16 = jnp.bfloat16
    zseq = jax.ShapeDtypeStruct((T_TOT, B, M), bf16)
    W_cat = jnp.concatenate([W_wk, W_wv, W_rk], axis=0)

    tk_all, zv_all, zrk_all = pl.pallas_call(
        _chains_body,
        out_shape=(zseq, zseq, zseq),
        grid=(B // BLK_A, S),
        in_specs=[
            pl.BlockSpec((1, BLK_A, I), lambda b, c: (c, b, 0)),
            pl.BlockSpec((E, I), lambda b, c: (0, 0)),
            pl.BlockSpec((E, E), lambda b, c: (0, 0)),
            pl.BlockSpec((3 * M, E), lambda b, c: (0, 0)),
        ],
        out_specs=(
            pl.BlockSpec((T, BLK_A, M), lambda b, c: (c, b, 0)),
            pl.BlockSpec((T, BLK_A, M), lambda b, c: (c, b, 0)),
            pl.BlockSpec((T, BLK_A, M), lambda b, c: (c, b, 0)),
        ),
        scratch_shapes=[
            pltpu.VMEM((T * BLK_A, E), f32),
            pltpu.VMEM((T * BLK_A, 3 * M), f32),
            pltpu.VMEM((3, BLK_A, E), f32),
            pltpu.VMEM((3, BLK_A, 3 * M), f32),
            pltpu.VMEM((BLK_A, M), f32),
        ],
        compiler_params=pltpu.CompilerParams(
            dimension_semantics=("parallel", "arbitrary"),
            vmem_limit_bytes=100 * 1024 * 1024,
        ),
        name="lif_chains",
    )(facts, W_emb, W_enc, W_cat)

    rv_all = pl.pallas_call(
        _memory_body,
        out_shape=zseq,
        grid=(B // BLK_B, T_TOT // CH_B),
        in_specs=[
            pl.BlockSpec((CH_B, BLK_B, M), lambda b, c: (c, b, 0)),
            pl.BlockSpec((CH_B, BLK_B, M), lambda b, c: (c, b, 0)),
            pl.BlockSpec((CH_B, BLK_B, M), lambda b, c: (c, b, 0)),
            pl.BlockSpec((BLK_B, M, M), lambda b, c: (b, 0, 0)),
        ],
        out_specs=pl.BlockSpec((CH_B, BLK_B, M), lambda b, c: (c, b, 0)),
        scratch_shapes=[
            pltpu.VMEM((BLK_B, M, M), f32),
            pltpu.VMEM((BLK_B, CH_B, M), bf16),
            pltpu.VMEM((BLK_B, CH_B, M), bf16),
            pltpu.VMEM((BLK_B, CH_B, M), bf16),
            pltpu.VMEM((BLK_B, CH_B, M), bf16),
        ],
        compiler_params=pltpu.CompilerParams(
            dimension_semantics=("parallel", "arbitrary"),
            vmem_limit_bytes=100 * 1024 * 1024,
        ),
        name="memory_read",
    )(tk_all, zv_all, zrk_all, mem0)

    out = pl.pallas_call(
        _readout_body,
        out_shape=jax.ShapeDtypeStruct((S, B, M), f32),
        grid=(B // BLK_C, S),
        in_specs=[pl.BlockSpec((T, BLK_C, M), lambda b, c: (c, b, 0))],
        out_specs=pl.BlockSpec((1, BLK_C, M), lambda b, c: (c, b, 0)),
        scratch_shapes=[pltpu.VMEM((3, BLK_C, M), f32),
                        pltpu.VMEM((BLK_C, M), f32)],
        compiler_params=pltpu.CompilerParams(
            dimension_semantics=("parallel", "arbitrary"),
            vmem_limit_bytes=100 * 1024 * 1024,
        ),
        name="readout",
    )(rv_all)

    return out


# speedup vs baseline: 1.0627x; 1.0128x over previous
import functools

import jax
import jax.numpy as jnp
from jax import lax
from jax.experimental import pallas as pl
from jax.experimental.pallas import tpu as pltpu

S, B, I = 50, 256, 200
E, M = 80, 256
T = 50
T_TOT = S * T
A_I, A_V = 0.9, 0.9
ETA = 0.01
LAM = 0.951229424500714

WIN_LO, WIN_HI = 19, 48


def _dotT(x, w):
    return lax.dot_general(x, w, (((x.ndim - 1,), (1,)), ((), ())))


BLK_A = 64


def _chains_body(facts_ref, wemb_ref, wenc_ref, wcat_ref,
                 tk_out, zv_out, zrk_out,
                 ze_s, xin_s,
                 enc_s, st_s, tk_s):
    c = pl.program_id(1)

    @pl.when(c == 0)
    def _():
        enc_s[...] = jnp.zeros_like(enc_s)
        st_s[...] = jnp.zeros_like(st_s)
        tk_s[...] = jnp.zeros_like(tk_s)

    x = facts_ref[0]
    emb = _dotT(x, wemb_ref[...])
    enc_in = _dotT(emb, wenc_ref[...])

    def p1(j, st):
        i, v, th = st
        i = A_I * i + enc_in
        v = A_V * v * (0.5 * (1.0 - th)) + i
        th = jnp.tanh(5.0 * v - 5.0)
        ze_s[pl.ds(j * BLK_A, BLK_A), :] = 0.5 + 0.5 * th
        return (i, v, th)

    enc_st = lax.fori_loop(
        0, T, p1, (enc_s[0], enc_s[1], enc_s[2]), unroll=10)
    enc_s[0] = enc_st[0]
    enc_s[1] = enc_st[1]
    enc_s[2] = enc_st[2]

    xin_s[...] = _dotT(ze_s[...], wcat_ref[...])

    def chain(lane0, out_ref, with_trace):
        def body(j, st):
            i, v, th, tk = st
            i = A_I * i + xin_s[pl.ds(j * BLK_A, BLK_A), lane0:lane0 + M]
            v = A_V * v * (0.5 * (1.0 - th)) + i
            th = jnp.tanh(5.0 * v - 5.0)
            z = 0.5 + 0.5 * th
            if with_trace:
                tk = LAM * tk + z
                out_ref[j] = tk.astype(jnp.bfloat16)
            else:
                out_ref[j] = z.astype(jnp.bfloat16)
            return (i, v, th, tk)

        st0 = (st_s[0, :, lane0:lane0 + M], st_s[1, :, lane0:lane0 + M],
               st_s[2, :, lane0:lane0 + M], tk_s[...])
        stf = lax.fori_loop(0, T, body, st0, unroll=25)
        st_s[0, :, lane0:lane0 + M] = stf[0]
        st_s[1, :, lane0:lane0 + M] = stf[1]
        st_s[2, :, lane0:lane0 + M] = stf[2]
        if with_trace:
            tk_s[...] = stf[3]

    chain(0, tk_out, True)
    chain(M, zv_out, False)
    chain(2 * M, zrk_out, False)


BLK_B = 32
CH_B = 125


def _memory_body(tk_ref, zv_ref, zrk_ref, mem0_ref, rv_out,
                 mem_s, tkT, zvT, zrkT, rvT):
    c = pl.program_id(1)

    @pl.when(c == 0)
    def _():
        mem_s[...] = mem0_ref[...]

    tkT[...] = jnp.swapaxes(tk_ref[...], 0, 1)
    zvT[...] = jnp.swapaxes(zv_ref[...], 0, 1)
    zrkT[...] = jnp.swapaxes(zrk_ref[...], 0, 1)

    row = lax.broadcasted_iota(jnp.int32, (CH_B, CH_B), 0)
    col = lax.broadcasted_iota(jnp.int32, (CH_B, CH_B), 1)
    causal = col <= row

    for be in range(BLK_B):
        tkb = tkT[be]
        zvb = zvT[be]
        zrkb = zrkT[be]
        memb = mem_s[be]
        g = lax.dot_general(zrkb, tkb, (((1,), (1,)), ((), ())),
                            preferred_element_type=jnp.float32)
        gm = jnp.where(causal, g, 0.0).astype(jnp.bfloat16)
        intra = lax.dot_general(gm, zvb, (((1,), (0,)), ((), ())),
                                preferred_element_type=jnp.float32)
        inter = lax.dot_general(zrkb, memb.astype(jnp.bfloat16),
                                (((1,), (1,)), ((), ())),
                                preferred_element_type=jnp.float32)
        rvT[be] = (inter + ETA * intra).astype(jnp.bfloat16)
        mem_s[be] = memb + ETA * lax.dot_general(
            zvb, tkb, (((0,), (0,)), ((), ())),
            preferred_element_type=jnp.float32)

    rv_out[...] = jnp.swapaxes(rvT[...], 0, 1)


BLK_C = 64


def _readout_body(rv_ref, out_ref, st_s, acc_s):
    c = pl.program_id(1)

    @pl.when(c == 0)
    def _():
        st_s[...] = jnp.zeros_like(st_s)

    acc_s[...] = jnp.zeros_like(acc_s)

    def step(j, st, accumulate):
        i, v, th = st
        i = A_I * i + rv_ref[j]
        v = A_V * v * (0.5 * (1.0 - th)) + i
        th = jnp.tanh(5.0 * v - 5.0)
        if accumulate:
            acc_s[...] = acc_s[...] + (0.5 + 0.5 * th)
        return (i, v, th)

    st = (st_s[0], st_s[1], st_s[2])
    st = lax.fori_loop(0, WIN_LO, functools.partial(step, accumulate=False),
                       st, unroll=5)
    st = lax.fori_loop(WIN_LO, WIN_HI + 1,
                       functools.partial(step, accumulate=True), st, unroll=5)
    st = lax.fori_loop(WIN_HI + 1, T, functools.partial(step, accumulate=False),
                       st, unroll=5)
    st_s[0] = st[0]
    st_s[1] = st[1]
    st_s[2] = st[2]

    out_ref[0] = acc_s[...]


def kernel(facts, mem0, W_emb, W_enc, W_wk, W_wv, W_rk):
    f32 = jnp.float32
    bf16 = jnp.bfloat16
    zseq = jax.ShapeDtypeStruct((T_TOT, B, M), bf16)
    W_cat = jnp.concatenate([W_wk, W_wv, W_rk], axis=0)

    tk_all, zv_all, zrk_all = pl.pallas_call(
        _chains_body,
        out_shape=(zseq, zseq, zseq),
        grid=(B // BLK_A, S),
        in_specs=[
            pl.BlockSpec((1, BLK_A, I), lambda b, c: (c, b, 0)),
            pl.BlockSpec((E, I), lambda b, c: (0, 0)),
            pl.BlockSpec((E, E), lambda b, c: (0, 0)),
            pl.BlockSpec((3 * M, E), lambda b, c: (0, 0)),
        ],
        out_specs=(
            pl.BlockSpec((T, BLK_A, M), lambda b, c: (c, b, 0)),
            pl.BlockSpec((T, BLK_A, M), lambda b, c: (c, b, 0)),
            pl.BlockSpec((T, BLK_A, M), lambda b, c: (c, b, 0)),
        ),
        scratch_shapes=[
            pltpu.VMEM((T * BLK_A, E), f32),
            pltpu.VMEM((T * BLK_A, 3 * M), f32),
            pltpu.VMEM((3, BLK_A, E), f32),
            pltpu.VMEM((3, BLK_A, 3 * M), f32),
            pltpu.VMEM((BLK_A, M), f32),
        ],
        compiler_params=pltpu.CompilerParams(
            dimension_semantics=("parallel", "arbitrary"),
            vmem_limit_bytes=100 * 1024 * 1024,
        ),
        name="lif_chains",
    )(facts, W_emb, W_enc, W_cat)

    rv_all = pl.pallas_call(
        _memory_body,
        out_shape=zseq,
        grid=(B // BLK_B, T_TOT // CH_B),
        in_specs=[
            pl.BlockSpec((CH_B, BLK_B, M), lambda b, c: (c, b, 0)),
            pl.BlockSpec((CH_B, BLK_B, M), lambda b, c: (c, b, 0)),
            pl.BlockSpec((CH_B, BLK_B, M), lambda b, c: (c, b, 0)),
            pl.BlockSpec((BLK_B, M, M), lambda b, c: (b, 0, 0)),
        ],
        out_specs=pl.BlockSpec((CH_B, BLK_B, M), lambda b, c: (c, b, 0)),
        scratch_shapes=[
            pltpu.VMEM((BLK_B, M, M), f32),
            pltpu.VMEM((BLK_B, CH_B, M), bf16),
            pltpu.VMEM((BLK_B, CH_B, M), bf16),
            pltpu.VMEM((BLK_B, CH_B, M), bf16),
            pltpu.VMEM((BLK_B, CH_B, M), bf16),
        ],
        compiler_params=pltpu.CompilerParams(
            dimension_semantics=("parallel", "arbitrary"),
            vmem_limit_bytes=100 * 1024 * 1024,
        ),
        name="memory_read",
    )(tk_all, zv_all, zrk_all, mem0)

    out = pl.pallas_call(
        _readout_body,
        out_shape=jax.ShapeDtypeStruct((S, B, M), f32),
        grid=(B // BLK_C, S),
        in_specs=[pl.BlockSpec((T, BLK_C, M), lambda b, c: (c, b, 0))],
        out_specs=pl.BlockSpec((1, BLK_C, M), lambda b, c: (c, b, 0)),
        scratch_shapes=[pltpu.VMEM((3, BLK_C, M), f32),
                        pltpu.VMEM((BLK_C, M), f32)],
        compiler_params=pltpu.CompilerParams(
            dimension_semantics=("parallel", "arbitrary"),
            vmem_limit_bytes=100 * 1024 * 1024,
        ),
        name="readout",
    )(rv_all)

    return out


# speedup vs baseline: 1.2125x; 1.1410x over previous
import functools

import jax
import jax.numpy as jnp
from jax import lax
from jax.experimental import pallas as pl
from jax.experimental.pallas import tpu as pltpu

S, B, I = 50, 256, 200
E, M = 80, 256
T = 50
T_TOT = S * T
A_I, A_V = 0.9, 0.9
ETA = 0.01
LAM = 0.951229424500714

WIN_LO, WIN_HI = 19, 48


def _dotT(x, w):
    return lax.dot_general(x, w, (((x.ndim - 1,), (1,)), ((), ())))


BLK_A = 64


def _chains_body(facts_ref, wemb_ref, wenc_ref, wcat_ref,
                 tk_out, zv_out, zrk_out,
                 ze_s, xin_s,
                 enc_s, st_s, tk_s):
    c = pl.program_id(1)

    @pl.when(c == 0)
    def _():
        enc_s[...] = jnp.zeros_like(enc_s)
        st_s[...] = jnp.zeros_like(st_s)
        tk_s[...] = jnp.zeros_like(tk_s)

    x = facts_ref[0]
    emb = _dotT(x, wemb_ref[...])
    enc_in = _dotT(emb, wenc_ref[...])

    def p1(j, st):
        i, v, th = st
        i = A_I * i + enc_in
        v = A_V * v * (0.5 * (1.0 - th)) + i
        th = jnp.tanh(5.0 * v - 5.0)
        ze_s[pl.ds(j * BLK_A, BLK_A), :] = 0.5 + 0.5 * th
        return (i, v, th)

    enc_st = lax.fori_loop(
        0, T, p1, (enc_s[0], enc_s[1], enc_s[2]), unroll=10)
    enc_s[0] = enc_st[0]
    enc_s[1] = enc_st[1]
    enc_s[2] = enc_st[2]

    xin_s[...] = _dotT(ze_s[...], wcat_ref[...])

    def chain(lane0, out_ref, with_trace):
        def body(j, st):
            i, v, th, tk = st
            i = A_I * i + xin_s[pl.ds(j * BLK_A, BLK_A), lane0:lane0 + M]
            v = A_V * v * (0.5 * (1.0 - th)) + i
            th = jnp.tanh(5.0 * v - 5.0)
            z = 0.5 + 0.5 * th
            if with_trace:
                tk = LAM * tk + z
                out_ref[j] = tk.astype(jnp.bfloat16)
            else:
                out_ref[j] = z.astype(jnp.bfloat16)
            return (i, v, th, tk)

        st0 = (st_s[0, :, lane0:lane0 + M], st_s[1, :, lane0:lane0 + M],
               st_s[2, :, lane0:lane0 + M], tk_s[...])
        stf = lax.fori_loop(0, T, body, st0, unroll=50)
        st_s[0, :, lane0:lane0 + M] = stf[0]
        st_s[1, :, lane0:lane0 + M] = stf[1]
        st_s[2, :, lane0:lane0 + M] = stf[2]
        if with_trace:
            tk_s[...] = stf[3]

    chain(0, tk_out, True)
    chain(M, zv_out, False)
    chain(2 * M, zrk_out, False)


BLK_B = 32
CH_B = 125


def _memory_body(tk_ref, zv_ref, zrk_ref, mem0_ref, rv_out,
                 mem_s, tkT, zvT, zrkT, rvT):
    c = pl.program_id(1)

    @pl.when(c == 0)
    def _():
        mem_s[...] = mem0_ref[...]

    tkT[...] = jnp.swapaxes(tk_ref[...], 0, 1)
    zvT[...] = jnp.swapaxes(zv_ref[...], 0, 1)
    zrkT[...] = jnp.swapaxes(zrk_ref[...], 0, 1)

    row = lax.broadcasted_iota(jnp.int32, (CH_B, CH_B), 0)
    col = lax.broadcasted_iota(jnp.int32, (CH_B, CH_B), 1)
    causal = col <= row

    for be in range(BLK_B):
        tkb = tkT[be]
        zvb = zvT[be]
        zrkb = zrkT[be]
        memb = mem_s[be]
        g = lax.dot_general(zrkb, tkb, (((1,), (1,)), ((), ())),
                            preferred_element_type=jnp.float32)
        gm = jnp.where(causal, g, 0.0).astype(jnp.bfloat16)
        intra = lax.dot_general(gm, zvb, (((1,), (0,)), ((), ())),
                                preferred_element_type=jnp.float32)
        inter = lax.dot_general(zrkb, memb.astype(jnp.bfloat16),
                                (((1,), (1,)), ((), ())),
                                preferred_element_type=jnp.float32)
        rvT[be] = (inter + ETA * intra).astype(jnp.bfloat16)
        mem_s[be] = memb + ETA * lax.dot_general(
            zvb, tkb, (((0,), (0,)), ((), ())),
            preferred_element_type=jnp.float32)

    rv_out[...] = jnp.swapaxes(rvT[...], 0, 1)


BLK_C = 64


def _readout_body(rv_ref, out_ref, st_s, acc_s):
    c = pl.program_id(1)

    @pl.when(c == 0)
    def _():
        st_s[...] = jnp.zeros_like(st_s)

    acc_s[...] = jnp.zeros_like(acc_s)

    def step(j, st, accumulate):
        i, v, th = st
        i = A_I * i + rv_ref[j]
        v = A_V * v * (0.5 * (1.0 - th)) + i
        th = jnp.tanh(5.0 * v - 5.0)
        if accumulate:
            acc_s[...] = acc_s[...] + (0.5 + 0.5 * th)
        return (i, v, th)

    st = (st_s[0], st_s[1], st_s[2])
    st = lax.fori_loop(0, WIN_LO, functools.partial(step, accumulate=False),
                       st, unroll=10)
    st = lax.fori_loop(WIN_LO, WIN_HI + 1,
                       functools.partial(step, accumulate=True), st, unroll=10)
    st = lax.fori_loop(WIN_HI + 1, T, functools.partial(step, accumulate=False),
                       st, unroll=10)
    st_s[0] = st[0]
    st_s[1] = st[1]
    st_s[2] = st[2]

    out_ref[0] = acc_s[...]


def kernel(facts, mem0, W_emb, W_enc, W_wk, W_wv, W_rk):
    f32 = jnp.float32
    bf16 = jnp.bfloat16
    zseq = jax.ShapeDtypeStruct((T_TOT, B, M), bf16)
    W_cat = jnp.concatenate([W_wk, W_wv, W_rk], axis=0)

    tk_all, zv_all, zrk_all = pl.pallas_call(
        _chains_body,
        out_shape=(zseq, zseq, zseq),
        grid=(B // BLK_A, S),
        in_specs=[
            pl.BlockSpec((1, BLK_A, I), lambda b, c: (c, b, 0)),
            pl.BlockSpec((E, I), lambda b, c: (0, 0)),
            pl.BlockSpec((E, E), lambda b, c: (0, 0)),
            pl.BlockSpec((3 * M, E), lambda b, c: (0, 0)),
        ],
        out_specs=(
            pl.BlockSpec((T, BLK_A, M), lambda b, c: (c, b, 0)),
            pl.BlockSpec((T, BLK_A, M), lambda b, c: (c, b, 0)),
            pl.BlockSpec((T, BLK_A, M), lambda b, c: (c, b, 0)),
        ),
        scratch_shapes=[
            pltpu.VMEM((T * BLK_A, E), f32),
            pltpu.VMEM((T * BLK_A, 3 * M), f32),
            pltpu.VMEM((3, BLK_A, E), f32),
            pltpu.VMEM((3, BLK_A, 3 * M), f32),
            pltpu.VMEM((BLK_A, M), f32),
        ],
        compiler_params=pltpu.CompilerParams(
            dimension_semantics=("parallel", "arbitrary"),
            vmem_limit_bytes=100 * 1024 * 1024,
        ),
        name="lif_chains",
    )(facts, W_emb, W_enc, W_cat)

    rv_all = pl.pallas_call(
        _memory_body,
        out_shape=zseq,
        grid=(B // BLK_B, T_TOT // CH_B),
        in_specs=[
            pl.BlockSpec((CH_B, BLK_B, M), lambda b, c: (c, b, 0)),
            pl.BlockSpec((CH_B, BLK_B, M), lambda b, c: (c, b, 0)),
            pl.BlockSpec((CH_B, BLK_B, M), lambda b, c: (c, b, 0)),
            pl.BlockSpec((BLK_B, M, M), lambda b, c: (b, 0, 0)),
        ],
        out_specs=pl.BlockSpec((CH_B, BLK_B, M), lambda b, c: (c, b, 0)),
        scratch_shapes=[
            pltpu.VMEM((BLK_B, M, M), f32),
            pltpu.VMEM((BLK_B, CH_B, M), bf16),
            pltpu.VMEM((BLK_B, CH_B, M), bf16),
            pltpu.VMEM((BLK_B, CH_B, M), bf16),
            pltpu.VMEM((BLK_B, CH_B, M), bf16),
        ],
        compiler_params=pltpu.CompilerParams(
            dimension_semantics=("parallel", "arbitrary"),
            vmem_limit_bytes=100 * 1024 * 1024,
        ),
        name="memory_read",
    )(tk_all, zv_all, zrk_all, mem0)

    out = pl.pallas_call(
        _readout_body,
        out_shape=jax.ShapeDtypeStruct((S, B, M), f32),
        grid=(B // BLK_C, S),
        in_specs=[pl.BlockSpec((T, BLK_C, M), lambda b, c: (c, b, 0))],
        out_specs=pl.BlockSpec((1, BLK_C, M), lambda b, c: (c, b, 0)),
        scratch_shapes=[pltpu.VMEM((3, BLK_C, M), f32),
                        pltpu.VMEM((BLK_C, M), f32)],
        compiler_params=pltpu.CompilerParams(
            dimension_semantics=("parallel", "arbitrary"),
            vmem_limit_bytes=100 * 1024 * 1024,
        ),
        name="readout",
    )(rv_all)

    return out
